# baseline (speedup 1.0000x reference)
import functools

import jax
import jax.numpy as jnp
from jax import lax
from jax.experimental import pallas as pl
from jax.experimental.pallas import tpu as pltpu
from jax.experimental.pallas import tpu_sc as plsc

DIM = 64
NCTX = 4
NNEG = 5
NC, NS, L = 2, 16, 16
NW = NC * NS
CHUNK = 64


def _sc_transpose_call(V):
    nb = V // 128
    rem = V - nb * 128
    per = nb // NW
    extra = nb % NW
    mesh = plsc.VectorSubcoreMesh(core_axis_name="c", subcore_axis_name="s")

    @functools.partial(
        pl.kernel,
        out_type=jax.ShapeDtypeStruct((V // 2, 128), jnp.float32),
        mesh=mesh,
        compiler_params=pltpu.CompilerParams(needs_layout_passes=False),
        scratch_types=[
            pltpu.VMEM((DIM, 128), jnp.float32),
            pltpu.VMEM((DIM, 128), jnp.float32),
            pltpu.VMEM((64, 128), jnp.float32),
            pltpu.VMEM((64, 128), jnp.float32),
            pltpu.SemaphoreType.DMA,
            pltpu.SemaphoreType.DMA,
            pltpu.SemaphoreType.DMA,
            pltpu.SemaphoreType.DMA,
        ],
    )
    def sc_transpose(wt_hbm, wtail_hbm, wp_hbm, tin0, tin1, tout0, tout1,
                     sin0, sin1, sout0, sout1):
        wid = lax.axis_index("s") * NC + lax.axis_index("c")
        lanes = lax.iota(jnp.int32, L)
        my_n = per + jnp.where(wid < extra, 1, 0)
        my_start = wid * per + jnp.minimum(wid, extra)
        bufs = ((tin0, tout0, sin0, sout0), (tin1, tout1, sin1, sout1))
        nmax = per + 1

        def fire_in(j, tin, sin):
            @pl.when(j < my_n)
            def _():
                pltpu.async_copy(
                    wt_hbm.at[:, pl.ds((my_start + j) * 128, 128)], tin, sin)

        def transpose_block(tin, tout, n_i):
            def diag_body(jj, carry3):
                iv0 = (lanes + jj) & (L - 1)
                q0 = lax.shift_right_logical(iv0, 1)
                hb = (iv0 & 1) << 6
                cols = [hb + db * L + lanes for db in range(DIM // L)]
                pending = None
                for ib2 in range(n_i // L // 2):
                    vals, metas = [], []
                    for ib in (2 * ib2, 2 * ib2 + 1):
                        iv = ib * L + iv0
                        q = ib * (L // 2) + q0
                        for db in range(DIM // L):
                            dvec = db * L + lanes
                            vals.append(plsc.load_gather(tin, [dvec, iv]))
                            metas.append((q, cols[db]))
                    if pending is not None:
                        for v, (q, col) in pending:
                            plsc.store_scatter(tout, [q, col], v)
                    pending = list(zip(vals, metas))
                for v, (q, col) in pending:
                    plsc.store_scatter(tout, [q, col], v)
                return carry3

            lax.fori_loop(0, L, diag_body, 0, unroll=2)

        for b in (0, 1):
            fire_in(b, bufs[b][0], bufs[b][2])

        def body(j2, carry):
            for b in (0, 1):
                j = 2 * j2 + b
                tin, tout, sin, sout = bufs[b]

                @pl.when(j < my_n)
                def _process():
                    pltpu.make_async_copy(
                        wt_hbm.at[:, pl.ds(0, 128)], tin, sin).wait()
                    transpose_block(tin, tout, 128)

                    @pl.when(j >= 2)
                    def _():
                        pltpu.make_async_copy(
                            tout, wp_hbm.at[pl.ds(0, 64)], sout).wait()
                    pltpu.async_copy(
                        tout, wp_hbm.at[pl.ds((my_start + j) * 64, 64)],
                        sout)
                    fire_in(j + 2, tin, sin)
            return carry

        lax.fori_loop(0, (nmax + 1) // 2, body, 0)
        for b in (0, 1):
            pltpu.make_async_copy(
                bufs[b][1], wp_hbm.at[pl.ds(0, 64)], bufs[b][3]).wait()

        if rem:
            @pl.when(wid == NW - 1)
            def _tail_block():
                pltpu.sync_copy(wtail_hbm, tout0.at[pl.ds(0, rem // 2)])
                pltpu.sync_copy(tout0.at[pl.ds(0, rem // 2)],
                                wp_hbm.at[pl.ds(nb * 64, rem // 2)])

    return sc_transpose


def _sc_scores_call(B):
    bpw = B // NW
    nchunk = bpw // CHUNK
    mesh = plsc.VectorSubcoreMesh(core_axis_name="c", subcore_axis_name="s")

    @functools.partial(
        pl.kernel,
        out_type=(jax.ShapeDtypeStruct((B,), jnp.float32),
                  jax.ShapeDtypeStruct((B,), jnp.float32)),
        mesh=mesh,
        compiler_params=pltpu.CompilerParams(needs_layout_passes=False),
        scratch_types=[
            pltpu.VMEM((CHUNK,), jnp.int32),
            pltpu.VMEM((CHUNK,), jnp.int32),
            pltpu.VMEM((CHUNK,), jnp.int32),
            pltpu.VMEM((NCTX * CHUNK,), jnp.int32),
            pltpu.VMEM((NCTX * CHUNK,), jnp.int32),
            pltpu.VMEM((NCTX * CHUNK,), jnp.int32),
            pltpu.VMEM((NNEG * CHUNK,), jnp.int32),
            pltpu.VMEM((NNEG * CHUNK,), jnp.int32),
            pltpu.VMEM((NNEG * CHUNK,), jnp.int32),
            pltpu.VMEM((CHUNK, 2 * DIM), jnp.float32),
            pltpu.VMEM((NCTX * CHUNK, 2 * DIM), jnp.float32),
            pltpu.VMEM((NNEG * CHUNK, 2 * DIM), jnp.float32),
            pltpu.VMEM((CHUNK,), jnp.float32),
            pltpu.VMEM((CHUNK,), jnp.float32),
            pltpu.SemaphoreType.DMA,
        ],
    )
    def sc_scores(cen_hbm, ctx_hbm, neg_hbm, wp_hbm, pos_hbm, negs_hbm,
                  cidx, cpair, choff, xidx, xpair, xhoff, nidx, npair, nhoff,
                  crows, xrows, nrows, pos_v, neg_v, sem):
        wid = lax.axis_index("s") * NC + lax.axis_index("c")
        lanes = lax.iota(jnp.int32, L)

        def split(src, dst_pair, dst_hoff, n):
            for m in range(n // L):
                v = src[pl.ds(m * L, L)]
                dst_pair[pl.ds(m * L, L)] = lax.shift_right_logical(v, 1)
                dst_hoff[pl.ds(m * L, L)] = (v & 1) << 6

        def chunk_body(g, carry):
            cb = wid * bpw + g * CHUNK

            pltpu.sync_copy(cen_hbm.at[pl.ds(cb, CHUNK)], cidx)
            pltpu.sync_copy(ctx_hbm.at[pl.ds(cb * NCTX, NCTX * CHUNK)], xidx)
            pltpu.sync_copy(neg_hbm.at[pl.ds(cb * NNEG, NNEG * CHUNK)], nidx)

            split(cidx, cpair, choff, CHUNK)
            split(xidx, xpair, xhoff, NCTX * CHUNK)
            split(nidx, npair, nhoff, NNEG * CHUNK)

            copies = [pltpu.async_copy(wp_hbm.at[cpair], crows, sem)]
            for j in range(NCTX):
                copies.append(pltpu.async_copy(
                    wp_hbm.at[xpair.at[pl.ds(j * CHUNK, CHUNK)]],
                    xrows.at[pl.ds(j * CHUNK, CHUNK)], sem))
            for j in range(NNEG):
                copies.append(pltpu.async_copy(
                    wp_hbm.at[npair.at[pl.ds(j * CHUNK, CHUNK)]],
                    nrows.at[pl.ds(j * CHUNK, CHUNK)], sem))
            for c in copies:
                c.wait()

            def group_body(t, carry2):
                bvec = t * L + lanes
                uo = choff[pl.ds(t * L, L)]
                xrow = [NCTX * bvec + k for k in range(NCTX)]
                nrow = [NNEG * bvec + k for k in range(NNEG)]
                xo = [plsc.load_gather(xhoff, [xrow[k]]) for k in range(NCTX)]
                no = [plsc.load_gather(nhoff, [nrow[k]]) for k in range(NNEG)]
                pos_acc = [jnp.zeros((L,), jnp.float32) for _ in range(2)]
                neg_acc = [jnp.zeros((L,), jnp.float32) for _ in range(2)]
                for d in range(DIM):
                    dv = (d + lanes) & (DIM - 1)
                    u = plsc.load_gather(crows, [bvec, uo + dv])
                    x = [plsc.load_gather(xrows, [xrow[k], xo[k] + dv])
                         for k in range(NCTX)]
                    n = [plsc.load_gather(nrows, [nrow[k], no[k] + dv])
                         for k in range(NNEG)]
                    xs = (x[0] + x[1]) + (x[2] + x[3])
                    ns = ((n[0] + n[1]) + (n[2] + n[3])) + n[4]
                    pos_acc[d & 1] = pos_acc[d & 1] + u * xs
                    neg_acc[d & 1] = neg_acc[d & 1] + u * ns
                pos_v[pl.ds(t * L, L)] = (pos_acc[0] + pos_acc[1]) * (
                    1.0 / NCTX)
                neg_v[pl.ds(t * L, L)] = (neg_acc[0] + neg_acc[1]) * (
                    1.0 / NNEG)
                return carry2

            lax.fori_loop(0, CHUNK // L, group_body, 0)

            pltpu.sync_copy(pos_v, pos_hbm.at[pl.ds(cb, CHUNK)])
            pltpu.sync_copy(neg_v, negs_hbm.at[pl.ds(cb, CHUNK)])
            return carry

        lax.fori_loop(0, nchunk, chunk_body, 0)

    return sc_scores


def _tail_body(pos_ref, neg_ref, out_ref):
    p = pos_ref[...]
    n = -neg_ref[...]
    lsp = jnp.minimum(p, 0.0) - jnp.log(1.0 + jnp.exp(-jnp.abs(p)))
    lsn = jnp.minimum(n, 0.0) - jnp.log(1.0 + jnp.exp(-jnp.abs(n)))
    b = pos_ref.shape[0] * pos_ref.shape[1]
    out_ref[...] = jnp.full((1, 1), -(jnp.sum(lsp) + jnp.sum(lsn)) / b,
                            jnp.float32)


def kernel(centers, context, neg_context, W):
    B = centers.shape[0]
    V = W.shape[0]
    nfull = (V // 128) * 128
    wtail = lax.slice(W, (nfull, 0), (V, DIM)).reshape((V - nfull) // 2,
                                                       2 * DIM)
    wp = _sc_transpose_call(V)(W.T, wtail)
    pos, neg = _sc_scores_call(B)(
        centers, context.reshape(-1), neg_context.reshape(-1), wp)

    rows = B // 128
    loss = pl.pallas_call(
        _tail_body,
        out_shape=jax.ShapeDtypeStruct((1, 1), jnp.float32),
    )(pos.reshape(rows, 128), neg.reshape(rows, 128))
    return loss[0, 0]


# speedup vs baseline: 1.0681x; 1.0681x over previous
import functools

import jax
import jax.numpy as jnp
from jax import lax
from jax.experimental import pallas as pl
from jax.experimental.pallas import tpu as pltpu
from jax.experimental.pallas import tpu_sc as plsc

DIM = 64
NCTX = 4
NNEG = 5
NC, NS, L = 2, 16, 16
NW = NC * NS
CHUNK = 64


def _sc_transpose_call(V):
    nb = V // 128
    rem = V - nb * 128
    per = nb // NW
    extra = nb % NW
    mesh = plsc.VectorSubcoreMesh(core_axis_name="c", subcore_axis_name="s")

    @functools.partial(
        pl.kernel,
        out_type=jax.ShapeDtypeStruct((V // 2, 128), jnp.float32),
        mesh=mesh,
        compiler_params=pltpu.CompilerParams(needs_layout_passes=False),
        scratch_types=[
            pltpu.VMEM((DIM, 128), jnp.float32),
            pltpu.VMEM((DIM, 128), jnp.float32),
            pltpu.VMEM((64, 128), jnp.float32),
            pltpu.VMEM((64, 128), jnp.float32),
            pltpu.SemaphoreType.DMA,
            pltpu.SemaphoreType.DMA,
            pltpu.SemaphoreType.DMA,
            pltpu.SemaphoreType.DMA,
        ],
    )
    def sc_transpose(wt_hbm, wtail_hbm, wp_hbm, tin0, tin1, tout0, tout1,
                     sin0, sin1, sout0, sout1):
        wid = lax.axis_index("s") * NC + lax.axis_index("c")
        lanes = lax.iota(jnp.int32, L)
        my_n = per + jnp.where(wid < extra, 1, 0)
        my_start = wid * per + jnp.minimum(wid, extra)
        bufs = ((tin0, tout0, sin0, sout0), (tin1, tout1, sin1, sout1))
        nmax = per + 1

        def fire_in(j, tin, sin):
            @pl.when(j < my_n)
            def _():
                pltpu.async_copy(
                    wt_hbm.at[:, pl.ds((my_start + j) * 128, 128)], tin, sin)

        def transpose_block(tin, tout, n_i):
            def diag_body(jj, carry3):
                iv0 = (lanes + jj) & (L - 1)
                q0 = lax.shift_right_logical(iv0, 1)
                hb = (iv0 & 1) << 6
                cols = [hb + db * L + lanes for db in range(DIM // L)]
                for ib2 in range(n_i // L // 2):
                    vals, metas = [], []
                    for ib in (2 * ib2, 2 * ib2 + 1):
                        iv = ib * L + iv0
                        q = ib * (L // 2) + q0
                        for db in range(DIM // L):
                            dvec = db * L + lanes
                            vals.append(plsc.load_gather(tin, [dvec, iv]))
                            metas.append((q, cols[db]))
                    for v, (q, col) in zip(vals, metas):
                        plsc.store_scatter(tout, [q, col], v)
                return carry3

            lax.fori_loop(0, L, diag_body, 0, unroll=2)

        for b in (0, 1):
            fire_in(b, bufs[b][0], bufs[b][2])

        def body(j2, carry):
            for b in (0, 1):
                j = 2 * j2 + b
                tin, tout, sin, sout = bufs[b]

                @pl.when(j < my_n)
                def _process():
                    pltpu.make_async_copy(
                        wt_hbm.at[:, pl.ds(0, 128)], tin, sin).wait()
                    transpose_block(tin, tout, 128)

                    @pl.when(j >= 2)
                    def _():
                        pltpu.make_async_copy(
                            tout, wp_hbm.at[pl.ds(0, 64)], sout).wait()
                    pltpu.async_copy(
                        tout, wp_hbm.at[pl.ds((my_start + j) * 64, 64)],
                        sout)
                    fire_in(j + 2, tin, sin)
            return carry

        lax.fori_loop(0, (nmax + 1) // 2, body, 0)
        for b in (0, 1):
            pltpu.make_async_copy(
                bufs[b][1], wp_hbm.at[pl.ds(0, 64)], bufs[b][3]).wait()

        if rem:
            @pl.when(wid == NW - 1)
            def _tail_block():
                pltpu.sync_copy(wtail_hbm, tout0.at[pl.ds(0, rem // 2)])
                pltpu.sync_copy(tout0.at[pl.ds(0, rem // 2)],
                                wp_hbm.at[pl.ds(nb * 64, rem // 2)])

    return sc_transpose


def _sc_scores_call(B):
    bpw = B // NW
    nchunk = bpw // CHUNK
    mesh = plsc.VectorSubcoreMesh(core_axis_name="c", subcore_axis_name="s")

    @functools.partial(
        pl.kernel,
        out_type=(jax.ShapeDtypeStruct((B,), jnp.float32),
                  jax.ShapeDtypeStruct((B,), jnp.float32)),
        mesh=mesh,
        compiler_params=pltpu.CompilerParams(needs_layout_passes=False),
        scratch_types=[
            pltpu.VMEM((CHUNK,), jnp.int32),
            pltpu.VMEM((CHUNK,), jnp.int32),
            pltpu.VMEM((CHUNK,), jnp.int32),
            pltpu.VMEM((NCTX * CHUNK,), jnp.int32),
            pltpu.VMEM((NCTX * CHUNK,), jnp.int32),
            pltpu.VMEM((NCTX * CHUNK,), jnp.int32),
            pltpu.VMEM((NNEG * CHUNK,), jnp.int32),
            pltpu.VMEM((NNEG * CHUNK,), jnp.int32),
            pltpu.VMEM((NNEG * CHUNK,), jnp.int32),
            pltpu.VMEM((CHUNK, 2 * DIM), jnp.float32),
            pltpu.VMEM((NCTX * CHUNK, 2 * DIM), jnp.float32),
            pltpu.VMEM((NNEG * CHUNK, 2 * DIM), jnp.float32),
            pltpu.VMEM((CHUNK,), jnp.float32),
            pltpu.VMEM((CHUNK,), jnp.float32),
            pltpu.SemaphoreType.DMA,
        ],
    )
    def sc_scores(cen_hbm, ctx_hbm, neg_hbm, wp_hbm, pos_hbm, negs_hbm,
                  cidx, cpair, choff, xidx, xpair, xhoff, nidx, npair, nhoff,
                  crows, xrows, nrows, pos_v, neg_v, sem):
        wid = lax.axis_index("s") * NC + lax.axis_index("c")
        lanes = lax.iota(jnp.int32, L)

        def split(src, dst_pair, dst_hoff, n):
            for m in range(n // L):
                v = src[pl.ds(m * L, L)]
                dst_pair[pl.ds(m * L, L)] = lax.shift_right_logical(v, 1)
                dst_hoff[pl.ds(m * L, L)] = (v & 1) << 6

        def chunk_body(g, carry):
            cb = wid * bpw + g * CHUNK

            pltpu.sync_copy(cen_hbm.at[pl.ds(cb, CHUNK)], cidx)
            pltpu.sync_copy(ctx_hbm.at[pl.ds(cb * NCTX, NCTX * CHUNK)], xidx)
            pltpu.sync_copy(neg_hbm.at[pl.ds(cb * NNEG, NNEG * CHUNK)], nidx)

            split(cidx, cpair, choff, CHUNK)
            split(xidx, xpair, xhoff, NCTX * CHUNK)
            split(nidx, npair, nhoff, NNEG * CHUNK)

            copies = [pltpu.async_copy(wp_hbm.at[cpair], crows, sem)]
            for j in range(NCTX):
                copies.append(pltpu.async_copy(
                    wp_hbm.at[xpair.at[pl.ds(j * CHUNK, CHUNK)]],
                    xrows.at[pl.ds(j * CHUNK, CHUNK)], sem))
            for j in range(NNEG):
                copies.append(pltpu.async_copy(
                    wp_hbm.at[npair.at[pl.ds(j * CHUNK, CHUNK)]],
                    nrows.at[pl.ds(j * CHUNK, CHUNK)], sem))
            for c in copies:
                c.wait()

            def group_body(t, carry2):
                bvec = t * L + lanes
                uo = choff[pl.ds(t * L, L)]
                xrow = [NCTX * bvec + k for k in range(NCTX)]
                nrow = [NNEG * bvec + k for k in range(NNEG)]
                xo = [plsc.load_gather(xhoff, [xrow[k]]) for k in range(NCTX)]
                no = [plsc.load_gather(nhoff, [nrow[k]]) for k in range(NNEG)]
                pos_acc = [jnp.zeros((L,), jnp.float32) for _ in range(2)]
                neg_acc = [jnp.zeros((L,), jnp.float32) for _ in range(2)]
                for d in range(DIM):
                    dv = (d + lanes) & (DIM - 1)
                    u = plsc.load_gather(crows, [bvec, uo + dv])
                    x = [plsc.load_gather(xrows, [xrow[k], xo[k] + dv])
                         for k in range(NCTX)]
                    n = [plsc.load_gather(nrows, [nrow[k], no[k] + dv])
                         for k in range(NNEG)]
                    xs = (x[0] + x[1]) + (x[2] + x[3])
                    ns = ((n[0] + n[1]) + (n[2] + n[3])) + n[4]
                    pos_acc[d & 1] = pos_acc[d & 1] + u * xs
                    neg_acc[d & 1] = neg_acc[d & 1] + u * ns
                pos_v[pl.ds(t * L, L)] = (pos_acc[0] + pos_acc[1]) * (
                    1.0 / NCTX)
                neg_v[pl.ds(t * L, L)] = (neg_acc[0] + neg_acc[1]) * (
                    1.0 / NNEG)
                return carry2

            lax.fori_loop(0, CHUNK // L, group_body, 0)

            pltpu.sync_copy(pos_v, pos_hbm.at[pl.ds(cb, CHUNK)])
            pltpu.sync_copy(neg_v, negs_hbm.at[pl.ds(cb, CHUNK)])
            return carry

        lax.fori_loop(0, nchunk, chunk_body, 0)

    return sc_scores


def _tail_body(pos_ref, neg_ref, out_ref):
    p = pos_ref[...]
    n = -neg_ref[...]
    lsp = jnp.minimum(p, 0.0) - jnp.log(1.0 + jnp.exp(-jnp.abs(p)))
    lsn = jnp.minimum(n, 0.0) - jnp.log(1.0 + jnp.exp(-jnp.abs(n)))
    b = pos_ref.shape[0] * pos_ref.shape[1]
    out_ref[...] = jnp.full((1, 1), -(jnp.sum(lsp) + jnp.sum(lsn)) / b,
                            jnp.float32)


def kernel(centers, context, neg_context, W):
    B = centers.shape[0]
    V = W.shape[0]
    nfull = (V // 128) * 128
    wtail = lax.slice(W, (nfull, 0), (V, DIM)).reshape((V - nfull) // 2,
                                                       2 * DIM)
    wp = _sc_transpose_call(V)(W.T, wtail)
    pos, neg = _sc_scores_call(B)(
        centers, context.reshape(-1), neg_context.reshape(-1), wp)

    rows = B // 128
    loss = pl.pallas_call(
        _tail_body,
        out_shape=jax.ShapeDtypeStruct((1, 1), jnp.float32),
    )(pos.reshape(rows, 128), neg.reshape(rows, 128))
    return loss[0, 0]
